# Initial kernel scaffold; baseline (speedup 1.0000x reference)
import jax
import jax.numpy as jnp
from jax.experimental import pallas as pl


def kernel(hidden_states, W_q, b_q, bn_gamma, bn_beta, sub_keys_0, sub_keys_1, expert_down, expert_up):
    raise NotImplementedError("write your pallas kernel here")


# speedup vs baseline: 93.3843x; 93.3843x over previous
import functools

import jax
import jax.numpy as jnp
from jax import lax
from jax.experimental import pallas as pl
from jax.experimental.pallas import tpu as pltpu
from jax.experimental.pallas import tpu_sc as plsc

B, S, D = 1, 2048, 768
H = 8
QD = 256
K = 8
PK0, PK1 = 128, 128
NE = PK0 * PK1
OD = 768
HK = H * K

TOK_TILE = 256
N_TILES = S // TOK_TILE


def _proj_body(x_ref, w_ref, b_ref, q_ref, st_ref):
    i = pl.program_id(0)
    q = jnp.dot(x_ref[...].astype(jnp.bfloat16),
                w_ref[...].astype(jnp.bfloat16),
                preferred_element_type=jnp.float32) + b_ref[...]
    q_ref[...] = q

    @pl.when(i == 0)
    def _():
        st_ref[...] = jnp.zeros_like(st_ref)

    st_ref[0:1, :] += jnp.sum(q, axis=0, keepdims=True)
    st_ref[1:2, :] += jnp.sum(q * q, axis=0, keepdims=True)


def _projection(x, W_q, b_q):
    return pl.pallas_call(
        _proj_body,
        grid=(N_TILES,),
        in_specs=[
            pl.BlockSpec((TOK_TILE, D), lambda i: (i, 0)),
            pl.BlockSpec((D, H * QD), lambda i: (0, 0)),
            pl.BlockSpec((1, H * QD), lambda i: (0, 0)),
        ],
        out_specs=[
            pl.BlockSpec((TOK_TILE, H * QD), lambda i: (i, 0)),
            pl.BlockSpec((8, H * QD), lambda i: (0, 0)),
        ],
        out_shape=[
            jax.ShapeDtypeStruct((S, H * QD), jnp.float32),
            jax.ShapeDtypeStruct((8, H * QD), jnp.float32),
        ],
    )(x, W_q, b_q.reshape(1, H * QD))


def _top8(x):
    n = x.shape[-1]
    iota = lax.broadcasted_iota(jnp.int32, x.shape, x.ndim - 1)
    vals, idxs = [], []
    for _ in range(K):
        m = jnp.max(x, axis=-1, keepdims=True)
        am = jnp.min(jnp.where(x == m, iota, n), axis=-1, keepdims=True)
        vals.append(m)
        idxs.append(am)
        x = jnp.where(iota == am, -jnp.inf, x)
    return jnp.concatenate(vals, axis=-1), jnp.concatenate(idxs, axis=-1)


def _top8_payload(cv, ce):
    vals, pays = [], []
    for _ in range(K):
        m = jnp.max(cv, axis=-1, keepdims=True)
        pe = jnp.min(jnp.where(cv == m, ce, NE), axis=-1, keepdims=True)
        vals.append(m)
        pays.append(pe)
        cv = jnp.where((cv == m) & (ce == pe), -jnp.inf, cv)
    return jnp.concatenate(vals, axis=-1), jnp.concatenate(pays, axis=-1)


def _route_body(q_ref, st_ref, g_ref, bta_ref, sk0_ref, sk1_ref,
                idx_ref, w_ref):
    st = st_ref[...]
    inv_n = 1.0 / float(S * H)
    mu = jnp.zeros((1, QD), jnp.float32)
    ex2 = jnp.zeros((1, QD), jnp.float32)
    for h in range(H):
        mu = mu + st[0:1, h * QD:(h + 1) * QD]
        ex2 = ex2 + st[1:2, h * QD:(h + 1) * QD]
    mu = mu * inv_n
    ex2 = ex2 * inv_n
    var = ex2 - mu * mu
    scale = lax.rsqrt(var + 1e-5) * g_ref[...]
    shift = bta_ref[...] - mu * scale

    def _l2n(a):
        n = jnp.sqrt(jnp.sum(a * a, axis=-1, keepdims=True))
        return a / jnp.maximum(n, 1e-12)

    sk0 = _l2n(sk0_ref[...])
    sk1 = _l2n(sk1_ref[...])

    idx_parts, w_parts = [], []
    for h in range(H):
        qh = q_ref[:, h * QD:(h + 1) * QD] * scale + shift
        q0 = _l2n(qh[:, :QD // 2])
        q1 = _l2n(qh[:, QD // 2:])
        s0 = lax.dot_general(q0.astype(jnp.bfloat16), sk0.astype(jnp.bfloat16),
                             (((1,), (1,)), ((), ())),
                             preferred_element_type=jnp.float32)
        s1 = lax.dot_general(q1.astype(jnp.bfloat16), sk1.astype(jnp.bfloat16),
                             (((1,), (1,)), ((), ())),
                             preferred_element_type=jnp.float32)
        v0, i0 = _top8(s0)
        v1, i1 = _top8(s1)
        cv = jnp.concatenate(
            [v0[:, a:a + 1] + v1 for a in range(K)], axis=-1)
        ce = jnp.concatenate(
            [i0[:, a:a + 1] * PK1 + i1 for a in range(K)], axis=-1)
        tv, te = _top8_payload(cv, ce)
        m = jnp.max(tv, axis=-1, keepdims=True)
        e = jnp.exp(tv - m)
        w = e / jnp.sum(e, axis=-1, keepdims=True)
        idx_parts.append(te)
        w_parts.append(w)
    idx_ref[...] = jnp.concatenate(idx_parts, axis=-1)
    w_ref[...] = jnp.concatenate(w_parts, axis=-1)


def _routing(q, st, bn_gamma, bn_beta, sk0, sk1):
    return pl.pallas_call(
        _route_body,
        grid=(N_TILES,),
        in_specs=[
            pl.BlockSpec((TOK_TILE, H * QD), lambda i: (i, 0)),
            pl.BlockSpec((8, H * QD), lambda i: (0, 0)),
            pl.BlockSpec((1, QD), lambda i: (0, 0)),
            pl.BlockSpec((1, QD), lambda i: (0, 0)),
            pl.BlockSpec((PK0, QD // 2), lambda i: (0, 0)),
            pl.BlockSpec((PK1, QD // 2), lambda i: (0, 0)),
        ],
        out_specs=[
            pl.BlockSpec((TOK_TILE, HK), lambda i: (i, 0)),
            pl.BlockSpec((TOK_TILE, HK), lambda i: (i, 0)),
        ],
        out_shape=[
            jax.ShapeDtypeStruct((S, HK), jnp.int32),
            jax.ShapeDtypeStruct((S, HK), jnp.float32),
        ],
    )(q, st, bn_gamma.reshape(1, QD), bn_beta.reshape(1, QD), sk0, sk1)


NW = 32
TPW = S // NW
DC = D // 16


def _erf(y):
    az = jnp.abs(y)
    t = 1.0 / (1.0 + 0.3275911 * az)
    poly = t * (0.254829592 + t * (-0.284496736 + t * (
        1.421413741 + t * (-1.453152027 + t * 1.061405429))))
    mag = 1.0 - poly * jnp.exp(-az * az)
    return jnp.where(y < 0.0, -mag, mag)


def _gelu16(z):
    return z * 0.5 * (1.0 + _erf(z * 0.7071067811865476))


def _sc_body(x_hbm, idx_hbm, w_hbm, dn_hbm, up_hbm, out_hbm,
             idx_v, w_v, x_v, dn_v, up_v, o_v, sem_d, sem_u):
    c = lax.axis_index("c")
    s = lax.axis_index("s")
    wid = s * 2 + c
    zero16 = jnp.zeros((16,), jnp.float32)
    iota16 = lax.iota(jnp.int32, 16)

    def tok(t, carry):
        row = wid * TPW + t
        pltpu.sync_copy(idx_hbm.at[row], idx_v)
        pltpu.sync_copy(w_hbm.at[row], w_v)
        pltpu.sync_copy(x_hbm.at[row], x_v)
        cp_d = pltpu.async_copy(dn_hbm.at[idx_v], dn_v, sem_d)
        cp_u = pltpu.async_copy(up_hbm.at[idx_v], up_v, sem_u)
        cp_d.wait()
        cp_u.wait()

        def zinit(j, _):
            o_v[pl.ds(j * 16, 16)] = zero16
            return 0

        lax.fori_loop(0, DC, zinit, 0)

        for kg in range(HK // 8):
            wchunk = w_v[pl.ds((kg // 2) * 16, 16)]

            def dj(j, accs, kg=kg):
                xj = x_v[pl.ds(j * 16, 16)]
                return tuple(
                    accs[i] + dn_v[kg * 8 + i, pl.ds(j * 16, 16)] * xj
                    for i in range(8))

            accs = lax.fori_loop(0, DC, dj, (zero16,) * 8)

            splats = []
            for i in range(8):
                lane = (kg % 2) * 8 + i
                ei = jnp.sum(accs[i])
                wk = jnp.sum(jnp.where(iota16 == lane, wchunk, 0.0))
                a = _gelu16(lax.broadcast_in_dim(ei, (16,), ())) \
                    * lax.broadcast_in_dim(wk, (16,), ())
                splats.append(a)

            def oj(j, _, kg=kg, splats=splats):
                acc = o_v[pl.ds(j * 16, 16)]
                for i in range(8):
                    acc = acc + splats[i] * up_v[kg * 8 + i, pl.ds(j * 16, 16)]
                o_v[pl.ds(j * 16, 16)] = acc
                return 0

            lax.fori_loop(0, DC, oj, 0)

        pltpu.sync_copy(o_v, out_hbm.at[row])
        return carry

    lax.fori_loop(0, TPW, tok, 0)


def _expert_sc(x, idx, w, expert_down, expert_up):
    mesh = plsc.VectorSubcoreMesh(core_axis_name="c", subcore_axis_name="s")
    f = functools.partial(
        pl.kernel,
        out_type=jax.ShapeDtypeStruct((S, OD), jnp.float32),
        mesh=mesh,
        compiler_params=pltpu.CompilerParams(needs_layout_passes=False),
        scratch_types=[
            pltpu.VMEM((HK,), jnp.int32),
            pltpu.VMEM((HK,), jnp.float32),
            pltpu.VMEM((D,), jnp.float32),
            pltpu.VMEM((HK, D), jnp.float32),
            pltpu.VMEM((HK, OD), jnp.float32),
            pltpu.VMEM((OD,), jnp.float32),
            pltpu.SemaphoreType.DMA,
            pltpu.SemaphoreType.DMA,
        ],
    )(_sc_body)
    return f(x, idx, w, expert_down, expert_up)


def kernel(hidden_states, W_q, b_q, bn_gamma, bn_beta, sub_keys_0, sub_keys_1,
           expert_down, expert_up):
    x = hidden_states.reshape(S, D)
    q, st = _projection(x, W_q, b_q)
    idx, w = _routing(q, st, bn_gamma, bn_beta, sub_keys_0, sub_keys_1)
    out = _expert_sc(x, idx, w, expert_down, expert_up)
    return out.reshape(B, S, OD)


# speedup vs baseline: 166.1528x; 1.7792x over previous
import functools

import jax
import jax.numpy as jnp
from jax import lax
from jax.experimental import pallas as pl
from jax.experimental.pallas import tpu as pltpu
from jax.experimental.pallas import tpu_sc as plsc

B, S, D = 1, 2048, 768
H = 8
QD = 256
K = 8
PK0, PK1 = 128, 128
NE = PK0 * PK1
OD = 768
HK = H * K

TOK_TILE = 256
N_TILES = S // TOK_TILE


def _proj_body(x_ref, w_ref, b_ref, q_ref, st_ref):
    i = pl.program_id(0)
    q = jnp.dot(x_ref[...].astype(jnp.bfloat16),
                w_ref[...].astype(jnp.bfloat16),
                preferred_element_type=jnp.float32) + b_ref[...]
    q_ref[...] = q

    @pl.when(i == 0)
    def _():
        st_ref[...] = jnp.zeros_like(st_ref)

    st_ref[0:1, :] += jnp.sum(q, axis=0, keepdims=True)
    st_ref[1:2, :] += jnp.sum(q * q, axis=0, keepdims=True)


def _projection(x, W_q, b_q):
    return pl.pallas_call(
        _proj_body,
        grid=(N_TILES,),
        in_specs=[
            pl.BlockSpec((TOK_TILE, D), lambda i: (i, 0)),
            pl.BlockSpec((D, H * QD), lambda i: (0, 0)),
            pl.BlockSpec((1, H * QD), lambda i: (0, 0)),
        ],
        out_specs=[
            pl.BlockSpec((TOK_TILE, H * QD), lambda i: (i, 0)),
            pl.BlockSpec((8, H * QD), lambda i: (0, 0)),
        ],
        out_shape=[
            jax.ShapeDtypeStruct((S, H * QD), jnp.float32),
            jax.ShapeDtypeStruct((8, H * QD), jnp.float32),
        ],
    )(x, W_q, b_q.reshape(1, H * QD))


def _top8(x):
    n = x.shape[-1]
    iota = lax.broadcasted_iota(jnp.int32, x.shape, x.ndim - 1)
    vals, idxs = [], []
    for _ in range(K):
        m = jnp.max(x, axis=-1, keepdims=True)
        am = jnp.min(jnp.where(x == m, iota, n), axis=-1, keepdims=True)
        vals.append(m)
        idxs.append(am)
        x = jnp.where(iota == am, -jnp.inf, x)
    return jnp.concatenate(vals, axis=-1), jnp.concatenate(idxs, axis=-1)


def _top8_payload(cv, ce):
    vals, pays = [], []
    for _ in range(K):
        m = jnp.max(cv, axis=-1, keepdims=True)
        pe = jnp.min(jnp.where(cv == m, ce, NE), axis=-1, keepdims=True)
        vals.append(m)
        pays.append(pe)
        cv = jnp.where((cv == m) & (ce == pe), -jnp.inf, cv)
    return jnp.concatenate(vals, axis=-1), jnp.concatenate(pays, axis=-1)


def _route_body(q_ref, st_ref, g_ref, bta_ref, sk0_ref, sk1_ref,
                idx_ref, w_ref):
    st = st_ref[...]
    inv_n = 1.0 / float(S * H)
    mu = jnp.zeros((1, QD), jnp.float32)
    ex2 = jnp.zeros((1, QD), jnp.float32)
    for h in range(H):
        mu = mu + st[0:1, h * QD:(h + 1) * QD]
        ex2 = ex2 + st[1:2, h * QD:(h + 1) * QD]
    mu = mu * inv_n
    ex2 = ex2 * inv_n
    var = ex2 - mu * mu
    scale = lax.rsqrt(var + 1e-5) * g_ref[...]
    shift = bta_ref[...] - mu * scale

    def _l2n(a):
        n = jnp.sqrt(jnp.sum(a * a, axis=-1, keepdims=True))
        return a / jnp.maximum(n, 1e-12)

    sk0 = _l2n(sk0_ref[...])
    sk1 = _l2n(sk1_ref[...])

    sk0b = sk0.astype(jnp.bfloat16)
    sk1b = sk1.astype(jnp.bfloat16)
    s0_parts, s1_parts = [], []
    for h in range(H):
        qh = q_ref[:, h * QD:(h + 1) * QD] * scale + shift
        q0 = _l2n(qh[:, :QD // 2]).astype(jnp.bfloat16)
        q1 = _l2n(qh[:, QD // 2:]).astype(jnp.bfloat16)
        s0 = lax.dot_general(q0, sk0b, (((1,), (1,)), ((), ())),
                             preferred_element_type=jnp.float32)
        s1 = lax.dot_general(q1, sk1b, (((1,), (1,)), ((), ())),
                             preferred_element_type=jnp.float32)
        s0_parts.append(s0.reshape(TOK_TILE, 1, PK0))
        s1_parts.append(s1.reshape(TOK_TILE, 1, PK1))
    v0, i0 = _top8(jnp.concatenate(s0_parts, axis=1))
    v1, i1 = _top8(jnp.concatenate(s1_parts, axis=1))
    cv = jnp.concatenate(
        [v0[:, :, a:a + 1] + v1 for a in range(K)], axis=-1)
    ce = jnp.concatenate(
        [i0[:, :, a:a + 1] * PK1 + i1 for a in range(K)], axis=-1)
    tv, te = _top8_payload(cv, ce)
    m = jnp.max(tv, axis=-1, keepdims=True)
    e = jnp.exp(tv - m)
    w = e / jnp.sum(e, axis=-1, keepdims=True)
    idx_ref[...] = te
    w_ref[...] = w


def _routing(q, st, bn_gamma, bn_beta, sk0, sk1):
    return pl.pallas_call(
        _route_body,
        grid=(N_TILES,),
        in_specs=[
            pl.BlockSpec((TOK_TILE, H * QD), lambda i: (i, 0)),
            pl.BlockSpec((8, H * QD), lambda i: (0, 0)),
            pl.BlockSpec((1, QD), lambda i: (0, 0)),
            pl.BlockSpec((1, QD), lambda i: (0, 0)),
            pl.BlockSpec((PK0, QD // 2), lambda i: (0, 0)),
            pl.BlockSpec((PK1, QD // 2), lambda i: (0, 0)),
        ],
        out_specs=[
            pl.BlockSpec((TOK_TILE, H, K), lambda i: (i, 0, 0)),
            pl.BlockSpec((TOK_TILE, H, K), lambda i: (i, 0, 0)),
        ],
        out_shape=[
            jax.ShapeDtypeStruct((S, H, K), jnp.int32),
            jax.ShapeDtypeStruct((S, H, K), jnp.float32),
        ],
    )(q, st, bn_gamma.reshape(1, QD), bn_beta.reshape(1, QD), sk0, sk1)


NW = 32
TPW = S // NW
DC = D // 16


def _erf(y):
    az = jnp.abs(y)
    t = 1.0 / (1.0 + 0.3275911 * az)
    poly = t * (0.254829592 + t * (-0.284496736 + t * (
        1.421413741 + t * (-1.453152027 + t * 1.061405429))))
    mag = 1.0 - poly * jnp.exp(-az * az)
    return jnp.where(y < 0.0, -mag, mag)


def _gelu16(z):
    return z * 0.5 * (1.0 + _erf(z * 0.7071067811865476))


CH = 32
NCHUNK = TPW * 2


def _sc_body(x_hbm, idx2_hbm, w_hbm, dn_hbm, up_hbm, out_hbm,
             idx_all, w_all, x_v, dn0, dn1, up0, up1, o_v,
             sd0, sd1, su0, su1, sx0, sx1):
    c = lax.axis_index("c")
    s = lax.axis_index("s")
    wid = s * 2 + c
    zero16 = jnp.zeros((16,), jnp.float32)
    iota16 = lax.iota(jnp.int32, 16)
    base_tok = wid * TPW
    base_chunk = wid * NCHUNK

    pltpu.sync_copy(idx2_hbm.at[pl.ds(base_chunk, NCHUNK)], idx_all)
    pltpu.sync_copy(w_hbm.at[pl.ds(base_tok, TPW)], w_all)

    def issue(tbl, cc, buf, sem):
        return pltpu.async_copy(tbl.at[idx_all.at[jnp.minimum(cc, NCHUNK - 1)]],
                                buf, sem)

    def drain(tbl, buf, sem):
        pltpu.make_async_copy(tbl.at[idx_all.at[0]], buf, sem).wait()

    def issue_x(tok, par, sem):
        return pltpu.async_copy(
            x_hbm.at[base_tok + jnp.minimum(tok, TPW - 1)], x_v.at[par], sem)

    def drain_x(par, sem):
        pltpu.make_async_copy(x_hbm.at[0], x_v.at[par], sem).wait()

    def chunk_compute(dnb, upb, t, xpar, half):
        for kg in range(4):
            gg = half * 2 + kg // 2
            wchunk = w_all[t, pl.ds(gg * 16, 16)]

            def dj(j, accs, kg=kg, xpar=xpar, dnb=dnb):
                xj = x_v[xpar, pl.ds(j * 16, 16)]
                return tuple(
                    accs[i] + dnb[kg * 8 + i, pl.ds(j * 16, 16)] * xj
                    for i in range(8))

            accs = lax.fori_loop(0, DC, dj, (zero16,) * 8)

            splats = []
            for i in range(8):
                lane = (kg % 2) * 8 + i
                ei = jnp.sum(accs[i])
                wk = jnp.sum(jnp.where(iota16 == lane, wchunk, 0.0))
                a = _gelu16(lax.broadcast_in_dim(ei, (16,), ())) \
                    * lax.broadcast_in_dim(wk, (16,), ())
                splats.append(a)

            first = (half == 0 and kg == 0)

            def oj(j, _, kg=kg, splats=splats, upb=upb, first=first):
                acc = zero16 if first else o_v[pl.ds(j * 16, 16)]
                for i in range(8):
                    acc = acc + splats[i] * upb[kg * 8 + i, pl.ds(j * 16, 16)]
                o_v[pl.ds(j * 16, 16)] = acc
                return 0

            lax.fori_loop(0, DC, oj, 0)

    issue(dn_hbm, 0, dn0, sd0)
    issue(up_hbm, 0, up0, su0)
    issue_x(0, 0, sx0)
    issue_x(1, 1, sx1)

    def pair(p, carry):
        t0 = 2 * p
        t1 = 2 * p + 1
        c0 = 4 * p
        issue(dn_hbm, c0 + 1, dn1, sd1)
        issue(up_hbm, c0 + 1, up1, su1)
        drain_x(0, sx0)
        drain(dn_hbm, dn0, sd0)
        drain(up_hbm, up0, su0)
        chunk_compute(dn0, up0, t0, 0, 0)
        issue(dn_hbm, c0 + 2, dn0, sd0)
        issue(up_hbm, c0 + 2, up0, su0)
        drain(dn_hbm, dn1, sd1)
        drain(up_hbm, up1, su1)
        chunk_compute(dn1, up1, t0, 0, 1)
        pltpu.sync_copy(o_v, out_hbm.at[base_tok + t0])
        issue_x(t0 + 2, 0, sx0)
        issue(dn_hbm, c0 + 3, dn1, sd1)
        issue(up_hbm, c0 + 3, up1, su1)
        drain_x(1, sx1)
        drain(dn_hbm, dn0, sd0)
        drain(up_hbm, up0, su0)
        chunk_compute(dn0, up0, t1, 1, 0)
        issue(dn_hbm, c0 + 4, dn0, sd0)
        issue(up_hbm, c0 + 4, up0, su0)
        drain(dn_hbm, dn1, sd1)
        drain(up_hbm, up1, su1)
        chunk_compute(dn1, up1, t1, 1, 1)
        pltpu.sync_copy(o_v, out_hbm.at[base_tok + t1])
        issue_x(t1 + 2, 1, sx1)
        return carry

    lax.fori_loop(0, TPW // 2, pair, 0)

    drain(dn_hbm, dn0, sd0)
    drain(up_hbm, up0, su0)
    drain_x(0, sx0)
    drain_x(1, sx1)


def _expert_sc(x, idx, w, expert_down, expert_up):
    mesh = plsc.VectorSubcoreMesh(core_axis_name="c", subcore_axis_name="s")
    idx2 = idx.reshape(S * 2, CH)
    f = functools.partial(
        pl.kernel,
        out_type=jax.ShapeDtypeStruct((S, OD), jnp.float32),
        mesh=mesh,
        compiler_params=pltpu.CompilerParams(needs_layout_passes=False),
        scratch_types=[
            pltpu.VMEM((NCHUNK, CH), jnp.int32),
            pltpu.VMEM((TPW, HK), jnp.float32),
            pltpu.VMEM((2, D), jnp.float32),
            pltpu.VMEM((CH, D), jnp.float32),
            pltpu.VMEM((CH, D), jnp.float32),
            pltpu.VMEM((CH, OD), jnp.float32),
            pltpu.VMEM((CH, OD), jnp.float32),
            pltpu.VMEM((OD,), jnp.float32),
            pltpu.SemaphoreType.DMA,
            pltpu.SemaphoreType.DMA,
            pltpu.SemaphoreType.DMA,
            pltpu.SemaphoreType.DMA,
            pltpu.SemaphoreType.DMA,
            pltpu.SemaphoreType.DMA,
        ],
    )(_sc_body)
    return f(x, idx2, w, expert_down, expert_up)


def kernel(hidden_states, W_q, b_q, bn_gamma, bn_beta, sub_keys_0, sub_keys_1,
           expert_down, expert_up):
    x = hidden_states.reshape(S, D)
    q, st = _projection(x, W_q, b_q)
    idx, w = _routing(q, st, bn_gamma, bn_beta, sub_keys_0, sub_keys_1)
    out = _expert_sc(x, idx.reshape(S, HK), w.reshape(S, HK),
                     expert_down, expert_up)
    return out.reshape(B, S, OD)
